# Initial kernel scaffold; baseline (speedup 1.0000x reference)
import jax
import jax.numpy as jnp
from jax.experimental import pallas as pl


def kernel(pos, batch, W1, g1, b1, W2, g2, b2, W3, g3, b3, W4, g4, b4, W5, g5, b5, W6, g6, b6, W7, bias7, g7, b7, W8, bias8):
    raise NotImplementedError("write your pallas kernel here")


# speedup vs baseline: 18.2369x; 18.2369x over previous
import functools

import jax
import jax.numpy as jnp
from jax import lax
from jax.experimental import pallas as pl
from jax.experimental.pallas import tpu as pltpu
from jax.experimental.pallas import tpu_sc as plsc

K = 16
P = 1024
B = 8
EPS = 1e-5

NUM_SC_CORES = 2
NUM_SUBCORES = 16
NUM_TILES = NUM_SC_CORES * NUM_SUBCORES
LANES = 16

CH = 64
PR = 256


def _lrelu(x):
    return jnp.where(x >= 0, x, 0.2 * x)


_ROWS = 256


def _knn_body(pr_ref, pc_ref, out_ref):
    rows = pr_ref[0]
    cols = pc_ref[0]
    d = ((rows[:, 0:1] - cols[0:1, :]) ** 2
         + (rows[:, 1:2] - cols[1:2, :]) ** 2
         + (rows[:, 2:3] - cols[2:3, :]) ** 2)
    col_iota = lax.broadcasted_iota(jnp.int32, (_ROWS, P), 1)
    big = jnp.int32(2**30)
    inf = jnp.float32(jnp.inf)
    for k in range(K):
        m = jnp.min(d, axis=1, keepdims=True)
        am = jnp.min(jnp.where(d <= m, col_iota, big), axis=1)
        out_ref[0, k, :] = am
        d = jnp.where(col_iota == am[:, None], inf, d)


def _knn(pos_r, pos_c):
    return pl.pallas_call(
        _knn_body,
        grid=(B, P // _ROWS),
        in_specs=[
            pl.BlockSpec((1, _ROWS, 3), lambda b, r: (b, r, 0)),
            pl.BlockSpec((1, 3, P), lambda b, r: (b, 0, 0)),
        ],
        out_specs=pl.BlockSpec((1, K, _ROWS), lambda b, r: (b, 0, r)),
        out_shape=jax.ShapeDtypeStruct((B, K, P), jnp.int32),
    )(pos_r, pos_c)


def _layer_mm(x_t, wd, wj, beta, cin, cout):
    def body(x_ref, wd_ref, wj_ref, bt_ref, a_ref, bm_ref):
        x = x_ref[0]
        if cin <= 4:
            a = jnp.zeros((cout, P), jnp.float32)
            bm = jnp.zeros((cout, P), jnp.float32)
            for c in range(cin):
                a = a + wd_ref[:, c:c + 1] * x[c:c + 1, :]
                bm = bm + wj_ref[:, c:c + 1] * x[c:c + 1, :]
        else:
            a = jnp.dot(wd_ref[...], x, preferred_element_type=jnp.float32)
            bm = jnp.dot(wj_ref[...], x, preferred_element_type=jnp.float32)
        a_ref[0] = a + bt_ref[...]
        bm_ref[0] = bm

    return pl.pallas_call(
        body,
        grid=(B,),
        in_specs=[
            pl.BlockSpec((1, cin, P), lambda b: (b, 0, 0)),
            pl.BlockSpec((cout, cin), lambda b: (0, 0)),
            pl.BlockSpec((cout, cin), lambda b: (0, 0)),
            pl.BlockSpec((cout, 1), lambda b: (0, 0)),
        ],
        out_specs=[
            pl.BlockSpec((1, cout, P), lambda b: (b, 0, 0)),
            pl.BlockSpec((1, cout, P), lambda b: (b, 0, 0)),
        ],
        out_shape=[
            jax.ShapeDtypeStruct((B, cout, P), jnp.float32),
            jax.ShapeDtypeStruct((B, cout, P), jnp.float32),
        ],
    )(x_t, wd, wj, beta)


_SC_CH = {64: 16, 128: 32, 256: 32}


def _sc_gather_max(a_t, bm_t, idx_flat, cout):
    ch_sz = _SC_CH[cout]
    nch = cout // ch_sz
    ntasks = B * nch
    tasks_per_tile = ntasks // NUM_TILES
    chp = ch_sz * P
    mesh = plsc.VectorSubcoreMesh(core_axis_name="c", subcore_axis_name="s")

    @functools.partial(
        pl.kernel,
        mesh=mesh,
        out_type=jax.ShapeDtypeStruct((B, cout * P), jnp.float32),
        scratch_types=[
            pltpu.VMEM((chp,), jnp.float32),
            pltpu.VMEM((chp,), jnp.float32),
            pltpu.VMEM((K * P,), jnp.int32),
            pltpu.VMEM((chp,), jnp.float32),
        ],
        compiler_params=pltpu.CompilerParams(needs_layout_passes=False),
    )
    def k(a_hbm, bm_hbm, idx_hbm, out_hbm, bm_v, a_v, idx_v, out_v):
        wid = lax.axis_index("s") * NUM_SC_CORES + lax.axis_index("c")
        lane = lax.iota(jnp.int32, LANES)
        for i in range(tasks_per_tile):
            t = wid * tasks_per_tile + i
            b = t // nch
            c0 = (t % nch) * chp
            pltpu.sync_copy(bm_hbm.at[b, pl.ds(c0, chp)], bm_v)
            pltpu.sync_copy(a_hbm.at[b, pl.ds(c0, chp)], a_v)
            pltpu.sync_copy(idx_hbm.at[b], idx_v)

            def pg_body(pg, _):
                pvec = pg * LANES + lane
                jv = [plsc.load_gather(idx_v, [kk * P + pvec])
                      for kk in range(K)]

                def c_body(c, _):
                    cbase = jnp.broadcast_to(c * P, (LANES,))
                    m = jnp.full((LANES,), -jnp.inf, jnp.float32)
                    for kk in range(K):
                        v = plsc.load_gather(bm_v, [cbase + jv[kk]])
                        m = jnp.maximum(m, v)
                    a = plsc.load_gather(a_v, [cbase + pvec])
                    o = _lrelu(m + a)
                    plsc.store_scatter(out_v, [cbase + pvec], o)
                    return 0

                lax.fori_loop(0, ch_sz, c_body, 0)
                return 0

            lax.fori_loop(0, P // LANES, pg_body, 0)
            pltpu.sync_copy(out_v, out_hbm.at[b, pl.ds(c0, chp)])

    return k(a_t.reshape(B, cout * P), bm_t.reshape(B, cout * P),
             idx_flat).reshape(B, cout, P)


def _conv5_pool(x1, x2, x3, x4, w5, b5):
    def body(x1_ref, x2_ref, x3_ref, x4_ref, w_ref, b_ref, out_ref):
        xcat = jnp.concatenate(
            [x1_ref[0], x2_ref[0], x3_ref[0], x4_ref[0]], axis=0)
        h = jnp.dot(w_ref[...], xcat, preferred_element_type=jnp.float32)
        h = _lrelu(h + b_ref[...])
        mx = jnp.max(h, axis=1)
        av = jnp.sum(h, axis=1) * jnp.float32(1.0 / P)
        out_ref[0, 0, :] = jnp.concatenate([mx, av], axis=0)

    return pl.pallas_call(
        body,
        grid=(B,),
        in_specs=[
            pl.BlockSpec((1, 64, P), lambda b: (b, 0, 0)),
            pl.BlockSpec((1, 64, P), lambda b: (b, 0, 0)),
            pl.BlockSpec((1, 128, P), lambda b: (b, 0, 0)),
            pl.BlockSpec((1, 256, P), lambda b: (b, 0, 0)),
            pl.BlockSpec((1024, 512), lambda b: (0, 0)),
            pl.BlockSpec((1024, 1), lambda b: (0, 0)),
        ],
        out_specs=pl.BlockSpec((1, 1, 2048), lambda b: (b, 0, 0)),
        out_shape=jax.ShapeDtypeStruct((B, 1, 2048), jnp.float32),
    )(x1, x2, x3, x4, w5, b5)


def _mlp_head(pooled, w6, b6, w7, b7, w8, b8):
    def body(p_ref, w6_ref, b6_ref, w7_ref, b7_ref, w8_ref, b8_ref, out_ref):
        dn = (((1,), (1,)), ((), ()))
        z = lax.dot_general(p_ref[...], w6_ref[...], dn,
                            preferred_element_type=jnp.float32)
        z = _lrelu(z + b6_ref[...])
        z = lax.dot_general(z, w7_ref[...], dn,
                            preferred_element_type=jnp.float32)
        z = _lrelu(z + b7_ref[...])
        z = lax.dot_general(z, w8_ref[...], dn,
                            preferred_element_type=jnp.float32)
        out_ref[...] = z + b8_ref[...]

    return pl.pallas_call(
        body,
        out_shape=jax.ShapeDtypeStruct((B, 40), jnp.float32),
    )(pooled, w6, b6, w7, b7, w8, b8)


def kernel(pos, batch, W1, g1, b1, W2, g2, b2, W3, g3, b3, W4, g4, b4,
           W5, g5, b5, W6, g6, b6, W7, bias7, g7, b7, W8, bias8):
    del batch
    pos_r = pos.reshape(B, P, 3)
    pos_c = jnp.transpose(pos_r, (0, 2, 1))

    idx_t = _knn(pos_r, pos_c)
    idx_flat = idx_t.reshape(B, K * P)

    inv = 1.0 / jnp.sqrt(jnp.float32(1.0 + EPS))

    def prep(W, g, bb, cin):
        ws = W * (g * inv)[:, None]
        wi, wj = ws[:, :cin], ws[:, cin:]
        return wi - wj, wj, bb[:, None]

    x_t = pos_c
    outs = []
    for (W, g, bb, cin, cout) in ((W1, g1, b1, 3, 64),
                                  (W2, g2, b2, 64, 64),
                                  (W3, g3, b3, 64, 128),
                                  (W4, g4, b4, 128, 256)):
        wd, wj, bt = prep(W, g, bb, cin)
        a_t, bm_t = _layer_mm(x_t, wd, wj, bt, cin, cout)
        x_t = _sc_gather_max(a_t, bm_t, idx_flat, cout)
        outs.append(x_t)

    w5s = W5 * (g5 * inv)[:, None]
    pooled = _conv5_pool(outs[0], outs[1], outs[2], outs[3], w5s,
                         b5[:, None]).reshape(B, 2048)

    w6s = W6 * (g6 * inv)[:, None]
    w7s = W7 * (g7 * inv)[:, None]
    b7s = (bias7 * g7 * inv + b7)
    return _mlp_head(pooled, w6s, b6[None, :], w7s, b7s[None, :],
                     W8, bias8[None, :])


# speedup vs baseline: 20.2272x; 1.1091x over previous
import functools

import jax
import jax.numpy as jnp
from jax import lax
from jax.experimental import pallas as pl
from jax.experimental.pallas import tpu as pltpu
from jax.experimental.pallas import tpu_sc as plsc

K = 16
P = 1024
B = 8
EPS = 1e-5

NUM_SC_CORES = 2
NUM_SUBCORES = 16
NUM_TILES = NUM_SC_CORES * NUM_SUBCORES
LANES = 16

CH = 64
PR = 256


def _lrelu(x):
    return jnp.where(x >= 0, x, 0.2 * x)


_ROWS = 256


def _knn_body(pr_ref, pc_ref, out_ref):
    rows = pr_ref[0]
    cols = pc_ref[0]
    d = ((rows[:, 0:1] - cols[0:1, :]) ** 2
         + (rows[:, 1:2] - cols[1:2, :]) ** 2
         + (rows[:, 2:3] - cols[2:3, :]) ** 2)
    col_iota = lax.broadcasted_iota(jnp.int32, (_ROWS, P), 1).astype(jnp.float32)
    big = jnp.float32(2.0**30)
    inf = jnp.float32(jnp.inf)
    for k in range(K):
        m = jnp.min(d, axis=1, keepdims=True)
        am = jnp.min(jnp.where(d <= m, col_iota, big), axis=1)
        out_ref[0, k, :] = am.astype(jnp.int32)
        d = jnp.where(col_iota == am[:, None], inf, d)


def _knn(pos_r, pos_c):
    return pl.pallas_call(
        _knn_body,
        grid=(B, P // _ROWS),
        in_specs=[
            pl.BlockSpec((1, _ROWS, 3), lambda b, r: (b, r, 0)),
            pl.BlockSpec((1, 3, P), lambda b, r: (b, 0, 0)),
        ],
        out_specs=pl.BlockSpec((1, K, _ROWS), lambda b, r: (b, 0, r)),
        out_shape=jax.ShapeDtypeStruct((B, K, P), jnp.int32),
    )(pos_r, pos_c)


def _layer_mm(x_t, wd, wj, beta, cin, cout):
    def body(x_ref, wd_ref, wj_ref, bt_ref, a_ref, bm_ref):
        x = x_ref[0]
        if cin <= 4:
            a = jnp.zeros((cout, P), jnp.float32)
            bm = jnp.zeros((cout, P), jnp.float32)
            for c in range(cin):
                a = a + wd_ref[:, c:c + 1] * x[c:c + 1, :]
                bm = bm + wj_ref[:, c:c + 1] * x[c:c + 1, :]
        else:
            a = jnp.dot(wd_ref[...], x, preferred_element_type=jnp.float32)
            bm = jnp.dot(wj_ref[...], x, preferred_element_type=jnp.float32)
        a_ref[0] = a + bt_ref[...]
        bm_ref[0] = bm

    return pl.pallas_call(
        body,
        grid=(B,),
        in_specs=[
            pl.BlockSpec((1, cin, P), lambda b: (b, 0, 0)),
            pl.BlockSpec((cout, cin), lambda b: (0, 0)),
            pl.BlockSpec((cout, cin), lambda b: (0, 0)),
            pl.BlockSpec((cout, 1), lambda b: (0, 0)),
        ],
        out_specs=[
            pl.BlockSpec((1, cout, P), lambda b: (b, 0, 0)),
            pl.BlockSpec((1, cout, P), lambda b: (b, 0, 0)),
        ],
        out_shape=[
            jax.ShapeDtypeStruct((B, cout, P), jnp.float32),
            jax.ShapeDtypeStruct((B, cout, P), jnp.float32),
        ],
    )(x_t, wd, wj, beta)


_SC_CH = {64: 16, 128: 32, 256: 32}


def _sc_gather_max(a_t, bm_t, idx_flat, cout):
    ch_sz = _SC_CH[cout]
    nch = cout // ch_sz
    ntasks = B * nch
    tasks_per_tile = ntasks // NUM_TILES
    chp = ch_sz * P
    mesh = plsc.VectorSubcoreMesh(core_axis_name="c", subcore_axis_name="s")

    @functools.partial(
        pl.kernel,
        mesh=mesh,
        out_type=jax.ShapeDtypeStruct((B, cout * P), jnp.float32),
        scratch_types=[
            pltpu.VMEM((chp,), jnp.float32),
            pltpu.VMEM((chp,), jnp.float32),
            pltpu.VMEM((K * P,), jnp.int32),
            pltpu.VMEM((chp,), jnp.float32),
        ],
        compiler_params=pltpu.CompilerParams(needs_layout_passes=False),
    )
    def k(a_hbm, bm_hbm, idx_hbm, out_hbm, bm_v, a_v, idx_v, out_v):
        wid = lax.axis_index("s") * NUM_SC_CORES + lax.axis_index("c")
        lane = lax.iota(jnp.int32, LANES)
        for i in range(tasks_per_tile):
            t = wid * tasks_per_tile + i
            b = t // nch
            c0 = (t % nch) * chp
            pltpu.sync_copy(bm_hbm.at[b, pl.ds(c0, chp)], bm_v)
            pltpu.sync_copy(a_hbm.at[b, pl.ds(c0, chp)], a_v)
            pltpu.sync_copy(idx_hbm.at[b], idx_v)

            def pg_body(pg, _):
                pvec = pg * LANES + lane
                jv = [plsc.load_gather(idx_v, [kk * P + pvec])
                      for kk in range(K)]

                def c_body(c, _):
                    cbase = jnp.broadcast_to(c * P, (LANES,))
                    m = jnp.full((LANES,), -jnp.inf, jnp.float32)
                    for kk in range(K):
                        v = plsc.load_gather(bm_v, [cbase + jv[kk]])
                        m = jnp.maximum(m, v)
                    a = plsc.load_gather(a_v, [cbase + pvec])
                    o = _lrelu(m + a)
                    plsc.store_scatter(out_v, [cbase + pvec], o)
                    return 0

                lax.fori_loop(0, ch_sz, c_body, 0)
                return 0

            lax.fori_loop(0, P // LANES, pg_body, 0)
            pltpu.sync_copy(out_v, out_hbm.at[b, pl.ds(c0, chp)])

    return k(a_t.reshape(B, cout * P), bm_t.reshape(B, cout * P),
             idx_flat).reshape(B, cout, P)


def _conv5_pool(x1, x2, x3, x4, w5, b5):
    def body(x1_ref, x2_ref, x3_ref, x4_ref, w_ref, b_ref, out_ref):
        xcat = jnp.concatenate(
            [x1_ref[0], x2_ref[0], x3_ref[0], x4_ref[0]], axis=0)
        h = jnp.dot(w_ref[...], xcat, preferred_element_type=jnp.float32)
        h = _lrelu(h + b_ref[...])
        mx = jnp.max(h, axis=1)
        av = jnp.sum(h, axis=1) * jnp.float32(1.0 / P)
        out_ref[0, 0, :] = jnp.concatenate([mx, av], axis=0)

    return pl.pallas_call(
        body,
        grid=(B,),
        in_specs=[
            pl.BlockSpec((1, 64, P), lambda b: (b, 0, 0)),
            pl.BlockSpec((1, 64, P), lambda b: (b, 0, 0)),
            pl.BlockSpec((1, 128, P), lambda b: (b, 0, 0)),
            pl.BlockSpec((1, 256, P), lambda b: (b, 0, 0)),
            pl.BlockSpec((1024, 512), lambda b: (0, 0)),
            pl.BlockSpec((1024, 1), lambda b: (0, 0)),
        ],
        out_specs=pl.BlockSpec((1, 1, 2048), lambda b: (b, 0, 0)),
        out_shape=jax.ShapeDtypeStruct((B, 1, 2048), jnp.float32),
    )(x1, x2, x3, x4, w5, b5)


def _mlp_head(pooled, w6, b6, w7, b7, w8, b8):
    def body(p_ref, w6_ref, b6_ref, w7_ref, b7_ref, w8_ref, b8_ref, out_ref):
        dn = (((1,), (1,)), ((), ()))
        z = lax.dot_general(p_ref[...], w6_ref[...], dn,
                            preferred_element_type=jnp.float32)
        z = _lrelu(z + b6_ref[...])
        z = lax.dot_general(z, w7_ref[...], dn,
                            preferred_element_type=jnp.float32)
        z = _lrelu(z + b7_ref[...])
        z = lax.dot_general(z, w8_ref[...], dn,
                            preferred_element_type=jnp.float32)
        out_ref[...] = z + b8_ref[...]

    return pl.pallas_call(
        body,
        out_shape=jax.ShapeDtypeStruct((B, 40), jnp.float32),
    )(pooled, w6, b6, w7, b7, w8, b8)


def kernel(pos, batch, W1, g1, b1, W2, g2, b2, W3, g3, b3, W4, g4, b4,
           W5, g5, b5, W6, g6, b6, W7, bias7, g7, b7, W8, bias8):
    del batch
    pos_r = pos.reshape(B, P, 3)
    pos_c = jnp.transpose(pos_r, (0, 2, 1))

    idx_t = _knn(pos_r, pos_c)
    idx_flat = idx_t.reshape(B, K * P)

    inv = 1.0 / jnp.sqrt(jnp.float32(1.0 + EPS))

    def prep(W, g, bb, cin):
        ws = W * (g * inv)[:, None]
        wi, wj = ws[:, :cin], ws[:, cin:]
        return wi - wj, wj, bb[:, None]

    x_t = pos_c
    outs = []
    for (W, g, bb, cin, cout) in ((W1, g1, b1, 3, 64),
                                  (W2, g2, b2, 64, 64),
                                  (W3, g3, b3, 64, 128),
                                  (W4, g4, b4, 128, 256)):
        wd, wj, bt = prep(W, g, bb, cin)
        a_t, bm_t = _layer_mm(x_t, wd, wj, bt, cin, cout)
        x_t = _sc_gather_max(a_t, bm_t, idx_flat, cout)
        outs.append(x_t)

    w5s = W5 * (g5 * inv)[:, None]
    pooled = _conv5_pool(outs[0], outs[1], outs[2], outs[3], w5s,
                         b5[:, None]).reshape(B, 2048)

    w6s = W6 * (g6 * inv)[:, None]
    w7s = W7 * (g7 * inv)[:, None]
    b7s = (bias7 * g7 * inv + b7)
    return _mlp_head(pooled, w6s, b6[None, :], w7s, b7s[None, :],
                     W8, bias8[None, :])


# speedup vs baseline: 22.5544x; 1.1151x over previous
import functools

import jax
import jax.numpy as jnp
from jax import lax
from jax.experimental import pallas as pl
from jax.experimental.pallas import tpu as pltpu
from jax.experimental.pallas import tpu_sc as plsc

K = 16
P = 1024
B = 8
EPS = 1e-5

NUM_SC_CORES = 2
NUM_SUBCORES = 16
NUM_TILES = NUM_SC_CORES * NUM_SUBCORES
LANES = 16


def _lrelu(x):
    return jnp.where(x >= 0, x, 0.2 * x)


_ROWS = 256


def _knn_body(pr_ref, pc_ref, out_ref):
    rows = pr_ref[0]
    cols = pc_ref[0]
    d = ((rows[:, 0:1] - cols[0:1, :]) ** 2
         + (rows[:, 1:2] - cols[1:2, :]) ** 2
         + (rows[:, 2:3] - cols[2:3, :]) ** 2)
    d = jnp.maximum(d, jnp.float32(2.0**-126))
    col_iota = lax.broadcasted_iota(jnp.int32, (_ROWS, P), 1)
    keys = lax.bitcast_convert_type(
        (lax.bitcast_convert_type(d, jnp.int32) & jnp.int32(~1023)) | col_iota,
        jnp.float32)
    inf = jnp.float32(jnp.inf)
    for k in range(K):
        m = jnp.min(keys, axis=1)
        out_ref[0, k, :] = (lax.bitcast_convert_type(m, jnp.int32)
                            & jnp.int32(1023))
        keys = jnp.where(keys == m[:, None], inf, keys)


def _knn(pos_r, pos_c):
    return pl.pallas_call(
        _knn_body,
        grid=(B, P // _ROWS),
        in_specs=[
            pl.BlockSpec((1, _ROWS, 3), lambda b, r: (b, r, 0)),
            pl.BlockSpec((1, 3, P), lambda b, r: (b, 0, 0)),
        ],
        out_specs=pl.BlockSpec((1, K, _ROWS), lambda b, r: (b, 0, r)),
        out_shape=jax.ShapeDtypeStruct((B, K, P), jnp.int32),
    )(pos_r, pos_c)


def _layer_mm(xa, xm, wd, wj, beta, cin, cout, fuse_in):
    def body(xa_ref, xm_ref, wd_ref, wj_ref, bt_ref, a_ref, bm_ref):
        if fuse_in:
            x = _lrelu(xa_ref[0] + xm_ref[0])
        else:
            x = xa_ref[0]
        if cin <= 4:
            a = jnp.zeros((cout, P), jnp.float32)
            bm = jnp.zeros((cout, P), jnp.float32)
            for c in range(cin):
                a = a + wd_ref[:, c:c + 1] * x[c:c + 1, :]
                bm = bm + wj_ref[:, c:c + 1] * x[c:c + 1, :]
        else:
            a = jnp.dot(wd_ref[...], x, preferred_element_type=jnp.float32)
            bm = jnp.dot(wj_ref[...], x, preferred_element_type=jnp.float32)
        a_ref[0] = a + bt_ref[...]
        bm_ref[0] = bm

    return pl.pallas_call(
        body,
        grid=(B,),
        in_specs=[
            pl.BlockSpec((1, cin, P), lambda b: (b, 0, 0)),
            pl.BlockSpec((1, cin, P), lambda b: (b, 0, 0)),
            pl.BlockSpec((cout, cin), lambda b: (0, 0)),
            pl.BlockSpec((cout, cin), lambda b: (0, 0)),
            pl.BlockSpec((cout, 1), lambda b: (0, 0)),
        ],
        out_specs=[
            pl.BlockSpec((1, cout, P), lambda b: (b, 0, 0)),
            pl.BlockSpec((1, cout, P), lambda b: (b, 0, 0)),
        ],
        out_shape=[
            jax.ShapeDtypeStruct((B, cout, P), jnp.float32),
            jax.ShapeDtypeStruct((B, cout, P), jnp.float32),
        ],
    )(xa, xm, wd, wj, beta)


_SC_CH = {64: 16, 128: 32, 256: 32}


def _sc_gather_max(bm_t, idx_t, cout):
    ch_sz = _SC_CH[cout]
    nch = cout // ch_sz
    ntasks = B * nch
    tasks_per_tile = ntasks // NUM_TILES
    mesh = plsc.VectorSubcoreMesh(core_axis_name="c", subcore_axis_name="s")

    @functools.partial(
        pl.kernel,
        mesh=mesh,
        out_type=jax.ShapeDtypeStruct((B, cout, P), jnp.float32),
        scratch_types=[
            pltpu.VMEM((ch_sz, P), jnp.float32),
            pltpu.VMEM((K, P), jnp.int32),
            pltpu.VMEM((ch_sz, P), jnp.float32),
        ],
        compiler_params=pltpu.CompilerParams(needs_layout_passes=False),
    )
    def k(bm_hbm, idx_hbm, out_hbm, bm_v, idx_v, out_v):
        wid = lax.axis_index("s") * NUM_SC_CORES + lax.axis_index("c")
        lane = lax.iota(jnp.int32, LANES)
        b = (wid * tasks_per_tile) // nch
        pltpu.sync_copy(idx_hbm.at[b], idx_v)
        for i in range(tasks_per_tile):
            t = wid * tasks_per_tile + i
            c0 = (t % nch) * ch_sz
            pltpu.sync_copy(bm_hbm.at[b, pl.ds(c0, ch_sz)], bm_v)

            def pg_body(pg, _):
                pvec = pg * LANES + lane
                jv = [plsc.load_gather(
                          idx_v, [jnp.full((LANES,), kk, jnp.int32), pvec])
                      for kk in range(K)]

                def c_body(c2, _):
                    for dc in range(2):
                        c = c2 * 2 + dc
                        cvec = jnp.broadcast_to(c, (LANES,)).astype(jnp.int32)
                        m = jnp.full((LANES,), -jnp.inf, jnp.float32)
                        for kk in range(K):
                            v = plsc.load_gather(bm_v, [cvec, jv[kk]])
                            m = jnp.maximum(m, v)
                        plsc.store_scatter(out_v, [cvec, pvec], m)
                    return 0

                lax.fori_loop(0, ch_sz // 2, c_body, 0)
                return 0

            lax.fori_loop(0, P // LANES, pg_body, 0)
            pltpu.sync_copy(out_v, out_hbm.at[b, pl.ds(c0, ch_sz)])

    return k(bm_t, idx_t)


def _conv5_pool(ams, w5, b5):
    def body(a1, m1, a2, m2, a3, m3, a4, m4, w_ref, b_ref, out_ref):
        xcat = jnp.concatenate(
            [_lrelu(a1[0] + m1[0]), _lrelu(a2[0] + m2[0]),
             _lrelu(a3[0] + m3[0]), _lrelu(a4[0] + m4[0])], axis=0)
        h = jnp.dot(w_ref[...], xcat, preferred_element_type=jnp.float32)
        h = _lrelu(h + b_ref[...])
        mx = jnp.max(h, axis=1)
        av = jnp.sum(h, axis=1) * jnp.float32(1.0 / P)
        out_ref[0, 0, :] = jnp.concatenate([mx, av], axis=0)

    specs = []
    for c in (64, 64, 128, 256):
        specs.append(pl.BlockSpec((1, c, P), lambda b: (b, 0, 0)))
        specs.append(pl.BlockSpec((1, c, P), lambda b: (b, 0, 0)))
    specs.append(pl.BlockSpec((1024, 512), lambda b: (0, 0)))
    specs.append(pl.BlockSpec((1024, 1), lambda b: (0, 0)))
    return pl.pallas_call(
        body,
        grid=(B,),
        in_specs=specs,
        out_specs=pl.BlockSpec((1, 1, 2048), lambda b: (b, 0, 0)),
        out_shape=jax.ShapeDtypeStruct((B, 1, 2048), jnp.float32),
    )(*ams, w5, b5)


def _mlp_head(pooled, w6, b6, w7, b7, w8, b8):
    def body(p_ref, w6_ref, b6_ref, w7_ref, b7_ref, w8_ref, b8_ref, out_ref):
        dn = (((1,), (1,)), ((), ()))
        z = lax.dot_general(p_ref[...], w6_ref[...], dn,
                            preferred_element_type=jnp.float32)
        z = _lrelu(z + b6_ref[...])
        z = lax.dot_general(z, w7_ref[...], dn,
                            preferred_element_type=jnp.float32)
        z = _lrelu(z + b7_ref[...])
        z = lax.dot_general(z, w8_ref[...], dn,
                            preferred_element_type=jnp.float32)
        out_ref[...] = z + b8_ref[...]

    return pl.pallas_call(
        body,
        out_shape=jax.ShapeDtypeStruct((B, 40), jnp.float32),
    )(pooled, w6, b6, w7, b7, w8, b8)


def kernel(pos, batch, W1, g1, b1, W2, g2, b2, W3, g3, b3, W4, g4, b4,
           W5, g5, b5, W6, g6, b6, W7, bias7, g7, b7, W8, bias8):
    del batch
    pos_r = pos.reshape(B, P, 3)
    pos_c = jnp.transpose(pos_r, (0, 2, 1))

    idx_t = _knn(pos_r, pos_c)

    inv = 1.0 / jnp.sqrt(jnp.float32(1.0 + EPS))

    def prep(W, g, bb, cin):
        ws = W * (g * inv)[:, None]
        wi, wj = ws[:, :cin], ws[:, cin:]
        return wi - wj, wj, bb[:, None]

    ams = []
    xa, xm = pos_c, pos_c
    fuse = False
    for (W, g, bb, cin, cout) in ((W1, g1, b1, 3, 64),
                                  (W2, g2, b2, 64, 64),
                                  (W3, g3, b3, 64, 128),
                                  (W4, g4, b4, 128, 256)):
        wd, wj, bt = prep(W, g, bb, cin)
        a_t, bm_t = _layer_mm(xa, xm, wd, wj, bt, cin, cout, fuse)
        m_t = _sc_gather_max(bm_t, idx_t, cout)
        ams.extend([a_t, m_t])
        xa, xm, fuse = a_t, m_t, True

    w5s = W5 * (g5 * inv)[:, None]
    pooled = _conv5_pool(ams, w5s, b5[:, None]).reshape(B, 2048)

    w6s = W6 * (g6 * inv)[:, None]
    w7s = W7 * (g7 * inv)[:, None]
    b7s = (bias7 * g7 * inv + b7)
    return _mlp_head(pooled, w6s, b6[None, :], w7s, b7s[None, :],
                     W8, bias8[None, :])
